# baseline (speedup 1.0000x reference)
import jax
import jax.numpy as jnp
from jax import lax
from jax.experimental import pallas as pl
from jax.experimental.pallas import tpu as pltpu

_NEG_T = 0.4
_POS_T = 0.5
_C = 2048


def _iou_block(pri_ref, bb_ref):
    pr = pri_ref[:, :]
    pcx, pcy = pr[0:1, :], pr[1:2, :]
    pw, ph = pr[2:3, :], pr[3:4, :]
    px1 = pcx - pw / 2
    py1 = pcy - ph / 2
    px2 = pcx + pw / 2
    py2 = pcy + ph / 2
    area_p = (px2 - px1) * (py2 - py1)

    gx1, gy1 = bb_ref[:, 0:1], bb_ref[:, 1:2]
    gx2, gy2 = bb_ref[:, 2:3], bb_ref[:, 3:4]
    area_g = (gx2 - gx1) * (gy2 - gy1)

    ltx = jnp.maximum(gx1, px1)
    lty = jnp.maximum(gy1, py1)
    rbx = jnp.minimum(gx2, px2)
    rby = jnp.minimum(gy2, py2)
    wx = jnp.maximum(rbx - ltx, 0.0)
    wy = jnp.maximum(rby - lty, 0.0)
    inter = wx * wy
    return inter / (area_g + area_p - inter)


def _pass1(nc, pri_ref, bb_ref, pm_ref, gbi_ref, gbv_s, gbx_s):
    i = pl.program_id(0)
    iou = _iou_block(pri_ref, bb_ref)

    pmax = jnp.max(iou, axis=0, keepdims=True)
    gidx = lax.broadcasted_iota(jnp.int32, iou.shape, 0)
    pam = jnp.min(jnp.where(iou == pmax, gidx, 128), axis=0, keepdims=True)
    pm_ref[0, 0:1, :] = pmax
    pm_ref[0, 1:2, :] = pam.astype(jnp.float32)

    cmax = jnp.max(iou, axis=1, keepdims=True)
    pidx = i * _C + lax.broadcasted_iota(jnp.int32, iou.shape, 1)
    cidx = jnp.min(jnp.where(iou == cmax, pidx, 2 ** 30), axis=1, keepdims=True)

    prev_v = jnp.where(i == 0, jnp.float32(-1.0), gbv_s[:, :])
    prev_i = jnp.where(i == 0, jnp.int32(0), gbx_s[:, :])
    upd = cmax > prev_v
    gbv_s[:, :] = jnp.where(upd, cmax, prev_v)
    gbx_s[:, :] = jnp.where(upd, cidx, prev_i)

    @pl.when(i == nc - 1)
    def _():
        gbi_ref[:, :] = gbx_s[:, :]


def _pass2(pri_ref, gt_ref, pm_ref, gbi_ref, out_ref):
    i = pl.program_id(0)
    pr = pri_ref[:, :]
    pcx, pcy = pr[0:1, :], pr[1:2, :]
    pw, ph = pr[2:3, :], pr[3:4, :]

    pmax = pm_ref[0, 0:1, :]
    pam = pm_ref[0, 1:2, :].astype(jnp.int32)
    gbi = gbi_ref[:, :]

    n = gbi.shape[0]
    c = pmax.shape[1]
    pidx = i * _C + lax.broadcasted_iota(jnp.int32, (n, c), 1)
    gidx = lax.broadcasted_iota(jnp.int32, (n, c), 0)

    forced = jnp.max(jnp.where(gbi == pidx, gidx, -1), axis=0, keepdims=True)
    emid = jnp.where(forced >= 0, forced, pam)
    emax = jnp.where(forced >= 0, jnp.float32(_POS_T), pmax)

    onehot = (gidx == emid).astype(jnp.float32)
    tx1, ty1 = gt_ref[0:1, :], gt_ref[1:2, :]
    tx2, ty2 = gt_ref[2:3, :], gt_ref[3:4, :]
    labf = gt_ref[4:5, :]
    tab = jnp.concatenate(
        [(tx1 + tx2) / 2, (ty1 + ty2) / 2, tx2 - tx1, ty2 - ty1, labf,
         jnp.zeros((3, n), jnp.float32)], axis=0)
    m = lax.dot_general(tab, onehot, (((1,), (0,)), ((), ())),
                        precision=lax.Precision.HIGHEST,
                        preferred_element_type=jnp.float32)
    mcx, mcy = m[0:1, :], m[1:2, :]
    mw, mh = m[2:3, :], m[3:4, :]
    mlab = m[4:5, :]

    dcx = (mcx - pcx) / pw / 0.1
    dcy = (mcy - pcy) / ph / 0.1
    dw = jnp.log(mw / pw) / 0.2
    dh = jnp.log(mh / ph) / 0.2

    lab = jnp.where(emax < _POS_T, jnp.float32(-1.0), mlab)
    lab = jnp.where(emax < _NEG_T, jnp.float32(0.0), lab)

    out_ref[:, :] = jnp.concatenate(
        [dcx, dcy, dw, dh, jnp.round(lab),
         jnp.zeros((3, c), jnp.float32)], axis=0)


def kernel(image, bboxes, labels, prior_boxes):
    p = prior_boxes.shape[0]
    n = bboxes.shape[0]
    nc = (p + _C - 1) // _C
    pp = nc * _C
    pad_n = pp - p

    pb_t = prior_boxes.T
    pad = jnp.concatenate(
        [jnp.full((2, pad_n), -4096.0, jnp.float32),
         jnp.ones((2, pad_n), jnp.float32)], axis=0)
    pri = jnp.concatenate([pb_t, pad], axis=1)
    pri8 = jnp.concatenate([pri, jnp.zeros((4, pp), jnp.float32)], axis=0)
    gt_t = jnp.concatenate(
        [bboxes.T, labels.astype(jnp.float32)[None, :],
         jnp.zeros((3, n), jnp.float32)], axis=0)

    fp32 = jnp.float32
    pm, gbi = pl.pallas_call(
        lambda *a: _pass1(nc, *a),
        grid=(nc,),
        in_specs=[
            pl.BlockSpec((8, _C), lambda i: (0, i)),
            pl.BlockSpec((n, 4), lambda i: (0, 0)),
        ],
        out_specs=[
            pl.BlockSpec((1, 8, _C), lambda i: (i, 0, 0)),
            pl.BlockSpec((n, 1), lambda i: (0, 0)),
        ],
        out_shape=[
            jax.ShapeDtypeStruct((nc, 8, _C), fp32),
            jax.ShapeDtypeStruct((n, 1), jnp.int32),
        ],
        scratch_shapes=[
            pltpu.VMEM((n, 1), fp32),
            pltpu.VMEM((n, 1), jnp.int32),
        ],
        compiler_params=pltpu.CompilerParams(
            dimension_semantics=("arbitrary",)),
    )(pri8, bboxes)

    outf = pl.pallas_call(
        _pass2,
        grid=(nc,),
        in_specs=[
            pl.BlockSpec((8, _C), lambda i: (0, i)),
            pl.BlockSpec((8, n), lambda i: (0, 0)),
            pl.BlockSpec((1, 8, _C), lambda i: (i, 0, 0)),
            pl.BlockSpec((n, 1), lambda i: (0, 0)),
        ],
        out_specs=pl.BlockSpec((8, _C), lambda i: (0, i)),
        out_shape=jax.ShapeDtypeStruct((8, pp), fp32),
        compiler_params=pltpu.CompilerParams(
            dimension_semantics=("arbitrary",)),
    )(pri8, gt_t, pm, gbi)

    offsets = outf[0:4, :p].T
    lab = outf[4, :p].astype(jnp.int32)
    return (image, offsets, lab)


# speedup vs baseline: 1.1586x; 1.1586x over previous
import jax
import jax.numpy as jnp
from jax import lax
from jax.experimental import pallas as pl
from jax.experimental.pallas import tpu as pltpu

_NEG_T = 0.4
_POS_T = 0.5
_C = 4096


def _iou_block(pri_ref, bb_ref):
    pr = pri_ref[:, :]
    pcx, pcy = pr[0:1, :], pr[1:2, :]
    pw, ph = pr[2:3, :], pr[3:4, :]
    px1 = pcx - pw / 2
    py1 = pcy - ph / 2
    px2 = pcx + pw / 2
    py2 = pcy + ph / 2
    area_p = (px2 - px1) * (py2 - py1)

    gx1, gy1 = bb_ref[:, 0:1], bb_ref[:, 1:2]
    gx2, gy2 = bb_ref[:, 2:3], bb_ref[:, 3:4]
    area_g = (gx2 - gx1) * (gy2 - gy1)

    ltx = jnp.maximum(gx1, px1)
    lty = jnp.maximum(gy1, py1)
    rbx = jnp.minimum(gx2, px2)
    rby = jnp.minimum(gy2, py2)
    wx = jnp.maximum(rbx - ltx, 0.0)
    wy = jnp.maximum(rby - lty, 0.0)
    inter = wx * wy
    return inter / (area_g + area_p - inter)


def _fused(pri_ref, bb_ref, gt_ref, out_ref, pm_s, gbv_s, gbx_s):
    t = pl.program_id(0)
    i = pl.program_id(1)

    @pl.when(t == 0)
    def _pass1():
        iou = _iou_block(pri_ref, bb_ref)

        pmax = jnp.max(iou, axis=0, keepdims=True)
        gidx = lax.broadcasted_iota(jnp.int32, iou.shape, 0)
        pam = jnp.min(jnp.where(iou == pmax, gidx, 128), axis=0, keepdims=True)
        pm_s[i, 0:1, :] = pmax
        pm_s[i, 1:2, :] = pam.astype(jnp.float32)

        cmax = jnp.max(iou, axis=1, keepdims=True)
        pidx = i * _C + lax.broadcasted_iota(jnp.int32, iou.shape, 1)
        cidx = jnp.min(jnp.where(iou == cmax, pidx, 2 ** 30), axis=1,
                       keepdims=True)

        prev_v = jnp.where(i == 0, jnp.float32(-1.0), gbv_s[:, :])
        prev_i = jnp.where(i == 0, jnp.int32(0), gbx_s[:, :])
        upd = cmax > prev_v
        gbv_s[:, :] = jnp.where(upd, cmax, prev_v)
        gbx_s[:, :] = jnp.where(upd, cidx, prev_i)

    @pl.when(t == 1)
    def _pass2():
        pr = pri_ref[:, :]
        pcx, pcy = pr[0:1, :], pr[1:2, :]
        pw, ph = pr[2:3, :], pr[3:4, :]

        pmax = pm_s[i, 0:1, :]
        pam = pm_s[i, 1:2, :].astype(jnp.int32)
        gbi = gbx_s[:, :]

        n = gbi.shape[0]
        c = pmax.shape[1]
        pidx = i * _C + lax.broadcasted_iota(jnp.int32, (n, c), 1)
        gidx = lax.broadcasted_iota(jnp.int32, (n, c), 0)

        forced = jnp.max(jnp.where(gbi == pidx, gidx, -1), axis=0,
                         keepdims=True)
        emid = jnp.where(forced >= 0, forced, pam)
        emax = jnp.where(forced >= 0, jnp.float32(_POS_T), pmax)

        onehot = (gidx == emid).astype(jnp.float32)
        tx1, ty1 = gt_ref[0:1, :], gt_ref[1:2, :]
        tx2, ty2 = gt_ref[2:3, :], gt_ref[3:4, :]
        labf = gt_ref[4:5, :]
        tab = jnp.concatenate(
            [(tx1 + tx2) / 2, (ty1 + ty2) / 2, tx2 - tx1, ty2 - ty1, labf,
             jnp.zeros((3, n), jnp.float32)], axis=0)
        m = lax.dot_general(tab, onehot, (((1,), (0,)), ((), ())),
                            precision=lax.Precision.HIGHEST,
                            preferred_element_type=jnp.float32)
        mcx, mcy = m[0:1, :], m[1:2, :]
        mw, mh = m[2:3, :], m[3:4, :]
        mlab = m[4:5, :]

        dcx = (mcx - pcx) / pw / 0.1
        dcy = (mcy - pcy) / ph / 0.1
        dw = jnp.log(mw / pw) / 0.2
        dh = jnp.log(mh / ph) / 0.2

        lab = jnp.where(emax < _POS_T, jnp.float32(-1.0), mlab)
        lab = jnp.where(emax < _NEG_T, jnp.float32(0.0), lab)

        out_ref[:, :] = jnp.concatenate(
            [dcx, dcy, dw, dh, jnp.round(lab),
             jnp.zeros((3, c), jnp.float32)], axis=0)


def kernel(image, bboxes, labels, prior_boxes):
    p = prior_boxes.shape[0]
    n = bboxes.shape[0]
    nc = (p + _C - 1) // _C
    pp = nc * _C
    pad_n = pp - p

    pb_t = prior_boxes.T
    pad = jnp.concatenate(
        [jnp.full((2, pad_n), -4096.0, jnp.float32),
         jnp.ones((2, pad_n), jnp.float32)], axis=0)
    pri = jnp.concatenate([pb_t, pad], axis=1)
    pri8 = jnp.concatenate([pri, jnp.zeros((4, pp), jnp.float32)], axis=0)
    gt_t = jnp.concatenate(
        [bboxes.T, labels.astype(jnp.float32)[None, :],
         jnp.zeros((3, n), jnp.float32)], axis=0)

    fp32 = jnp.float32
    outf = pl.pallas_call(
        _fused,
        grid=(2, nc),
        in_specs=[
            pl.BlockSpec((8, _C), lambda t, i: (0, i)),
            pl.BlockSpec((n, 4), lambda t, i: (0, 0)),
            pl.BlockSpec((8, n), lambda t, i: (0, 0)),
        ],
        out_specs=pl.BlockSpec((8, _C), lambda t, i: (0, i * t)),
        out_shape=jax.ShapeDtypeStruct((8, pp), fp32),
        scratch_shapes=[
            pltpu.VMEM((nc, 8, _C), fp32),
            pltpu.VMEM((n, 1), fp32),
            pltpu.VMEM((n, 1), jnp.int32),
        ],
        compiler_params=pltpu.CompilerParams(
            dimension_semantics=("arbitrary", "arbitrary")),
    )(pri8, bboxes, gt_t)

    offsets = outf[0:4, :p].T
    lab = outf[4, :p].astype(jnp.int32)
    return (image, offsets, lab)


# speedup vs baseline: 1.3099x; 1.1306x over previous
import jax
import jax.numpy as jnp
from jax import lax
from jax.experimental import pallas as pl
from jax.experimental.pallas import tpu as pltpu

_NEG_T = 0.4
_POS_T = 0.5
_C = 4096


def _iou_block(pri_ref, bb_ref):
    pr = pri_ref[:, :]
    pcx, pcy = pr[0:1, :], pr[1:2, :]
    pw, ph = pr[2:3, :], pr[3:4, :]
    px1 = pcx - pw / 2
    py1 = pcy - ph / 2
    px2 = pcx + pw / 2
    py2 = pcy + ph / 2
    area_p = (px2 - px1) * (py2 - py1)

    gx1, gy1 = bb_ref[:, 0:1], bb_ref[:, 1:2]
    gx2, gy2 = bb_ref[:, 2:3], bb_ref[:, 3:4]
    area_g = (gx2 - gx1) * (gy2 - gy1)

    ltx = jnp.maximum(gx1, px1)
    lty = jnp.maximum(gy1, py1)
    rbx = jnp.minimum(gx2, px2)
    rby = jnp.minimum(gy2, py2)
    wx = jnp.maximum(rbx - ltx, 0.0)
    wy = jnp.maximum(rby - lty, 0.0)
    inter = wx * wy
    return inter / (area_g + area_p - inter)


def _fused(pri_ref, bb_ref, gt_ref, out_ref, pm_s, gbv_s, gbx_s):
    t = pl.program_id(0)
    i = pl.program_id(1)

    @pl.when(t == 0)
    def _pass1():
        iou = _iou_block(pri_ref, bb_ref)

        pmax = jnp.max(iou, axis=0, keepdims=True)
        gidx = lax.broadcasted_iota(jnp.int32, iou.shape, 0)
        pam = jnp.min(jnp.where(iou == pmax, gidx, 128), axis=0, keepdims=True)
        pm_s[i, 0:1, :] = pmax
        pm_s[i, 1:2, :] = pam.astype(jnp.float32)

        cmax = jnp.max(iou, axis=1, keepdims=True)
        pidx = i * _C + lax.broadcasted_iota(jnp.int32, iou.shape, 1)
        cidx = jnp.min(jnp.where(iou == cmax, pidx, 2 ** 30), axis=1,
                       keepdims=True)

        prev_v = jnp.where(i == 0, jnp.float32(-1.0), gbv_s[:, :])
        prev_i = jnp.where(i == 0, jnp.int32(0), gbx_s[:, :])
        upd = cmax > prev_v
        gbv_s[:, :] = jnp.where(upd, cmax, prev_v)
        gbx_s[:, :] = jnp.where(upd, cidx, prev_i)

    @pl.when(t == 1)
    def _pass2():
        pr = pri_ref[:, :]
        pcx, pcy = pr[0:1, :], pr[1:2, :]
        pw, ph = pr[2:3, :], pr[3:4, :]

        pmax = pm_s[i, 0:1, :]
        pam = pm_s[i, 1:2, :].astype(jnp.int32)
        gbi = gbx_s[:, :]

        n = gbi.shape[0]
        c = pmax.shape[1]
        pidx = i * _C + lax.broadcasted_iota(jnp.int32, (n, c), 1)
        gidx = lax.broadcasted_iota(jnp.int32, (n, c), 0)

        forced = jnp.max(jnp.where(gbi == pidx, gidx, -1), axis=0,
                         keepdims=True)
        emid = jnp.where(forced >= 0, forced, pam)
        emax = jnp.where(forced >= 0, jnp.float32(_POS_T), pmax)

        onehot = (gidx == emid).astype(jnp.float32)
        tx1, ty1 = gt_ref[0:1, :], gt_ref[1:2, :]
        tx2, ty2 = gt_ref[2:3, :], gt_ref[3:4, :]
        labf = gt_ref[4:5, :]
        tab = jnp.concatenate(
            [(tx1 + tx2) / 2, (ty1 + ty2) / 2, tx2 - tx1, ty2 - ty1, labf,
             jnp.zeros((3, n), jnp.float32)], axis=0)
        dims = (((1,), (0,)), ((), ()))
        tab_hi = tab.astype(jnp.bfloat16).astype(jnp.float32)
        tab_lo = tab - tab_hi
        m = (lax.dot_general(tab_hi, onehot, dims,
                             preferred_element_type=jnp.float32)
             + lax.dot_general(tab_lo, onehot, dims,
                               preferred_element_type=jnp.float32))
        mcx, mcy = m[0:1, :], m[1:2, :]
        mw, mh = m[2:3, :], m[3:4, :]
        mlab = m[4:5, :]

        dcx = (mcx - pcx) / pw / 0.1
        dcy = (mcy - pcy) / ph / 0.1
        dw = jnp.log(mw / pw) / 0.2
        dh = jnp.log(mh / ph) / 0.2

        lab = jnp.where(emax < _POS_T, jnp.float32(-1.0), mlab)
        lab = jnp.where(emax < _NEG_T, jnp.float32(0.0), lab)

        out_ref[:, :] = jnp.concatenate(
            [dcx, dcy, dw, dh, jnp.round(lab),
             jnp.zeros((3, c), jnp.float32)], axis=0)


def kernel(image, bboxes, labels, prior_boxes):
    p = prior_boxes.shape[0]
    n = bboxes.shape[0]
    nc = (p + _C - 1) // _C
    pp = nc * _C
    pad_n = pp - p

    pb_t = prior_boxes.T
    pad = jnp.concatenate(
        [jnp.full((2, pad_n), -4096.0, jnp.float32),
         jnp.ones((2, pad_n), jnp.float32)], axis=0)
    pri = jnp.concatenate([pb_t, pad], axis=1)
    pri8 = jnp.concatenate([pri, jnp.zeros((4, pp), jnp.float32)], axis=0)
    gt_t = jnp.concatenate(
        [bboxes.T, labels.astype(jnp.float32)[None, :],
         jnp.zeros((3, n), jnp.float32)], axis=0)

    fp32 = jnp.float32
    outf = pl.pallas_call(
        _fused,
        grid=(2, nc),
        in_specs=[
            pl.BlockSpec((8, _C), lambda t, i: (0, i)),
            pl.BlockSpec((n, 4), lambda t, i: (0, 0)),
            pl.BlockSpec((8, n), lambda t, i: (0, 0)),
        ],
        out_specs=pl.BlockSpec((8, _C), lambda t, i: (0, i * t)),
        out_shape=jax.ShapeDtypeStruct((8, pp), fp32),
        scratch_shapes=[
            pltpu.VMEM((nc, 8, _C), fp32),
            pltpu.VMEM((n, 1), fp32),
            pltpu.VMEM((n, 1), jnp.int32),
        ],
        compiler_params=pltpu.CompilerParams(
            dimension_semantics=("arbitrary", "arbitrary")),
    )(pri8, bboxes, gt_t)

    offsets = outf[0:4, :p].T
    lab = outf[4, :p].astype(jnp.int32)
    return (image, offsets, lab)


# speedup vs baseline: 1.3850x; 1.0573x over previous
import jax
import jax.numpy as jnp
from jax import lax
from jax.experimental import pallas as pl
from jax.experimental.pallas import tpu as pltpu

_NEG_T = 0.4
_POS_T = 0.5
_C = 8192


def _iou_block(pri_ref, bb_ref):
    pr = pri_ref[:, :]
    pcx, pcy = pr[0:1, :], pr[1:2, :]
    pw, ph = pr[2:3, :], pr[3:4, :]
    px1 = pcx - pw / 2
    py1 = pcy - ph / 2
    px2 = pcx + pw / 2
    py2 = pcy + ph / 2
    area_p = (px2 - px1) * (py2 - py1)

    gx1, gy1 = bb_ref[:, 0:1], bb_ref[:, 1:2]
    gx2, gy2 = bb_ref[:, 2:3], bb_ref[:, 3:4]
    area_g = (gx2 - gx1) * (gy2 - gy1)

    ltx = jnp.maximum(gx1, px1)
    lty = jnp.maximum(gy1, py1)
    rbx = jnp.minimum(gx2, px2)
    rby = jnp.minimum(gy2, py2)
    wx = jnp.maximum(rbx - ltx, 0.0)
    wy = jnp.maximum(rby - lty, 0.0)
    inter = wx * wy
    return inter / (area_g + area_p - inter)


def _fused(pri_ref, bb_ref, gt_ref, out_ref, pm_s, gbv_s, gbx_s):
    t = pl.program_id(0)
    i = pl.program_id(1)

    @pl.when(t == 0)
    def _pass1():
        iou = _iou_block(pri_ref, bb_ref)

        pmax = jnp.max(iou, axis=0, keepdims=True)
        gidx = lax.broadcasted_iota(jnp.int32, iou.shape, 0)
        pam = jnp.min(jnp.where(iou == pmax, gidx, 128), axis=0, keepdims=True)
        pm_s[i, 0:1, :] = pmax
        pm_s[i, 1:2, :] = pam.astype(jnp.float32)

        cmax = jnp.max(iou, axis=1, keepdims=True)
        pidx = i * _C + lax.broadcasted_iota(jnp.int32, iou.shape, 1)
        cidx = jnp.min(jnp.where(iou == cmax, pidx, 2 ** 30), axis=1,
                       keepdims=True)

        prev_v = jnp.where(i == 0, jnp.float32(-1.0), gbv_s[:, :])
        prev_i = jnp.where(i == 0, jnp.int32(0), gbx_s[:, :])
        upd = cmax > prev_v
        gbv_s[:, :] = jnp.where(upd, cmax, prev_v)
        gbx_s[:, :] = jnp.where(upd, cidx, prev_i)

    @pl.when(t == 1)
    def _pass2():
        pr = pri_ref[:, :]
        pcx, pcy = pr[0:1, :], pr[1:2, :]
        pw, ph = pr[2:3, :], pr[3:4, :]

        pmax = pm_s[i, 0:1, :]
        pam = pm_s[i, 1:2, :].astype(jnp.int32)
        gbi = gbx_s[:, :]

        n = gbi.shape[0]
        c = pmax.shape[1]
        pidx = i * _C + lax.broadcasted_iota(jnp.int32, (n, c), 1)
        gidx = lax.broadcasted_iota(jnp.int32, (n, c), 0)

        forced = jnp.max(jnp.where(gbi == pidx, gidx, -1), axis=0,
                         keepdims=True)
        emid = jnp.where(forced >= 0, forced, pam)
        emax = jnp.where(forced >= 0, jnp.float32(_POS_T), pmax)

        onehot = (gidx == emid).astype(jnp.float32)
        tx1, ty1 = gt_ref[0:1, :], gt_ref[1:2, :]
        tx2, ty2 = gt_ref[2:3, :], gt_ref[3:4, :]
        labf = gt_ref[4:5, :]
        tab = jnp.concatenate(
            [(tx1 + tx2) / 2, (ty1 + ty2) / 2, tx2 - tx1, ty2 - ty1, labf,
             jnp.zeros((3, n), jnp.float32)], axis=0)
        dims = (((1,), (0,)), ((), ()))
        tab_hi = tab.astype(jnp.bfloat16).astype(jnp.float32)
        tab_lo = tab - tab_hi
        m = (lax.dot_general(tab_hi, onehot, dims,
                             preferred_element_type=jnp.float32)
             + lax.dot_general(tab_lo, onehot, dims,
                               preferred_element_type=jnp.float32))
        mcx, mcy = m[0:1, :], m[1:2, :]
        mw, mh = m[2:3, :], m[3:4, :]
        mlab = m[4:5, :]

        dcx = (mcx - pcx) / pw / 0.1
        dcy = (mcy - pcy) / ph / 0.1
        dw = jnp.log(mw / pw) / 0.2
        dh = jnp.log(mh / ph) / 0.2

        lab = jnp.where(emax < _POS_T, jnp.float32(-1.0), mlab)
        lab = jnp.where(emax < _NEG_T, jnp.float32(0.0), lab)

        out_ref[:, :] = jnp.concatenate(
            [dcx, dcy, dw, dh, jnp.round(lab),
             jnp.zeros((3, c), jnp.float32)], axis=0)


def kernel(image, bboxes, labels, prior_boxes):
    p = prior_boxes.shape[0]
    n = bboxes.shape[0]
    nc = (p + _C - 1) // _C
    pp = nc * _C
    pad_n = pp - p

    pb_t = prior_boxes.T
    pad = jnp.concatenate(
        [jnp.full((2, pad_n), -4096.0, jnp.float32),
         jnp.ones((2, pad_n), jnp.float32)], axis=0)
    pri = jnp.concatenate([pb_t, pad], axis=1)
    pri8 = jnp.concatenate([pri, jnp.zeros((4, pp), jnp.float32)], axis=0)
    gt_t = jnp.concatenate(
        [bboxes.T, labels.astype(jnp.float32)[None, :],
         jnp.zeros((3, n), jnp.float32)], axis=0)

    fp32 = jnp.float32
    outf = pl.pallas_call(
        _fused,
        grid=(2, nc),
        in_specs=[
            pl.BlockSpec((8, _C), lambda t, i: (0, i)),
            pl.BlockSpec((n, 4), lambda t, i: (0, 0)),
            pl.BlockSpec((8, n), lambda t, i: (0, 0)),
        ],
        out_specs=pl.BlockSpec((8, _C), lambda t, i: (0, i * t)),
        out_shape=jax.ShapeDtypeStruct((8, pp), fp32),
        scratch_shapes=[
            pltpu.VMEM((nc, 8, _C), fp32),
            pltpu.VMEM((n, 1), fp32),
            pltpu.VMEM((n, 1), jnp.int32),
        ],
        compiler_params=pltpu.CompilerParams(
            dimension_semantics=("arbitrary", "arbitrary")),
    )(pri8, bboxes, gt_t)

    offsets = outf[0:4, :p].T
    lab = outf[4, :p].astype(jnp.int32)
    return (image, offsets, lab)
